# baseline (speedup 1.0000x reference)
import functools

import jax
import jax.numpy as jnp
from jax import lax
from jax.experimental import pallas as pl
from jax.experimental.pallas import tpu as pltpu
from jax.experimental.pallas import tpu_sc as plsc

NC = 2
NS = 16
ECHUNK = 128
F = 128


def _sc_mesh():
  return plsc.VectorSubcoreMesh(
      core_axis_name="c", subcore_axis_name="s",
      num_cores=NC, num_subcores=NS)


def _make_deg_kernel(n_pad, n_chunks):
  rows_per_sub = n_pad // NS

  @functools.partial(
      pl.kernel, mesh=_sc_mesh(),
      out_type=jax.ShapeDtypeStruct((NC, n_pad, F), jnp.float32),
      scratch_types=[
          pltpu.VMEM((n_chunks, ECHUNK), jnp.int32),
          pltpu.VMEM((ECHUNK, F), jnp.float32),
          pltpu.VMEM_SHARED((n_pad, F), jnp.float32),
      ],
  )
  def k(dsts_hbm, zeros_hbm, ones_hbm, out_hbm, dst_v, ones_v, acc_s):
    c = lax.axis_index("c")
    s = lax.axis_index("s")
    pltpu.sync_copy(dsts_hbm.at[c, s], dst_v)
    pltpu.sync_copy(ones_hbm, ones_v)
    sl = pl.ds(s * rows_per_sub, rows_per_sub)
    pltpu.sync_copy(zeros_hbm.at[sl], acc_s.at[sl])
    plsc.subcore_barrier()

    @pl.loop(0, n_chunks)
    def _(j):
      pltpu.sync_copy(ones_v, acc_s.at[dst_v.at[j]], add=True)

    plsc.subcore_barrier()
    pltpu.sync_copy(acc_s.at[sl], out_hbm.at[c, sl])

  return k


NBUF = 2
NPHASE = 2


def _make_edge_scatter_kernel(n_pad, n_chunks):
  rows_per_sub = n_pad // NS
  assert n_chunks % (NBUF * NPHASE) == 0
  ph_chunks = n_chunks // NPHASE

  @functools.partial(
      pl.kernel, mesh=_sc_mesh(),
      out_type=jax.ShapeDtypeStruct((NC, n_pad, F), jnp.float32),
      scratch_types=[
          pltpu.VMEM((ph_chunks, ECHUNK), jnp.int32),
          pltpu.VMEM((ph_chunks, ECHUNK), jnp.int32),
          pltpu.VMEM((NBUF, ECHUNK, F), jnp.float32),
          pltpu.VMEM_SHARED((n_pad, F), jnp.float32),
          pltpu.SemaphoreType.DMA,
      ],
  )
  def k(g_hbm, srcs_hbm, dsts_hbm, zeros_hbm, out_hbm,
        src_v, dst_v, bufs_v, acc_s, sem):
    c = lax.axis_index("c")
    s = lax.axis_index("s")
    sl = pl.ds(s * rows_per_sub, rows_per_sub)
    pltpu.sync_copy(zeros_hbm.at[sl], acc_s.at[sl])
    plsc.subcore_barrier()

    for ph in range(NPHASE):
      pltpu.sync_copy(srcs_hbm.at[c, s, pl.ds(ph * ph_chunks, ph_chunks)],
                      src_v)
      pltpu.sync_copy(dsts_hbm.at[c, s, pl.ds(ph * ph_chunks, ph_chunks)],
                      dst_v)

      @pl.loop(0, ph_chunks // NBUF)
      def _(grp):
        j0 = grp * NBUF
        handles = [
            pltpu.async_copy(g_hbm.at[src_v.at[j0 + b]], bufs_v.at[b], sem)
            for b in range(NBUF)
        ]
        for h in handles:
          h.wait()
        for b in range(NBUF):
          pltpu.sync_copy(bufs_v.at[b], acc_s.at[dst_v.at[j0 + b]], add=True)

    plsc.subcore_barrier()
    pltpu.sync_copy(acc_s.at[sl], out_hbm.at[c, sl])

  return k


def _make_gather_kernel(n_rows_out, n_chunks):
  rows_per_tile = n_chunks * ECHUNK
  assert n_chunks % 2 == 0

  @functools.partial(
      pl.kernel, mesh=_sc_mesh(),
      out_type=jax.ShapeDtypeStruct((n_rows_out, F), jnp.float32),
      scratch_types=[
          pltpu.VMEM((n_chunks, ECHUNK), jnp.int32),
          pltpu.VMEM((2, ECHUNK, F), jnp.float32),
          pltpu.SemaphoreType.DMA,
      ],
  )
  def k(z_hbm, idx_hbm, out_hbm, idx_v, bufs_v, sem):
    c = lax.axis_index("c")
    s = lax.axis_index("s")
    base = (c * NS + s) * rows_per_tile
    pltpu.sync_copy(idx_hbm.at[c, s], idx_v)

    @pl.loop(0, n_chunks // 2)
    def _(grp):
      j0 = grp * 2
      handles = [
          pltpu.async_copy(z_hbm.at[idx_v.at[j0 + b]], bufs_v.at[b], sem)
          for b in range(2)
      ]
      for h in handles:
        h.wait()
      for b in range(2):
        pltpu.sync_copy(
            bufs_v.at[b], out_hbm.at[pl.ds(base + (j0 + b) * ECHUNK, ECHUNK)])

  return k


def _dot(a, b):
  return jnp.dot(a, b, preferred_element_type=jnp.float32,
                 precision=lax.Precision.HIGHEST)


def _make_layer1_tc(n_nodes, n_pad, blk):
  grid = (n_nodes // blk,)

  def body(degp_ref, x_ref, w_ref, dinv_ref, g_ref):
    deg = degp_ref[0] + degp_ref[1] + 1.0
    dinv = lax.rsqrt(deg)
    h = _dot(x_ref[...], w_ref[...])
    dinv_ref[...] = dinv
    g_ref[...] = h * dinv

  return pl.pallas_call(
      body,
      grid=grid,
      in_specs=[
          pl.BlockSpec((NC, blk, F), lambda i: (0, i, 0)),
          pl.BlockSpec((blk, F), lambda i: (i, 0)),
          pl.BlockSpec((F, F), lambda i: (0, 0)),
      ],
      out_specs=[
          pl.BlockSpec((blk, F), lambda i: (i, 0)),
          pl.BlockSpec((blk, F), lambda i: (i, 0)),
      ],
      out_shape=[
          jax.ShapeDtypeStruct((n_nodes, F), jnp.float32),
          jax.ShapeDtypeStruct((n_nodes, F), jnp.float32),
      ],
  )


def _make_combine_tc(n_nodes, blk, with_matmul):
  grid = (n_nodes // blk,)

  if with_matmul:
    def body(accp_ref, g_ref, dinv_ref, b_ref, w_ref, out_ref):
      dinv = dinv_ref[...]
      z = (accp_ref[0] + accp_ref[1] + g_ref[...]) * dinv + b_ref[...]
      z = jnp.maximum(z, 0.0)
      out_ref[...] = _dot(z, w_ref[...]) * dinv
    in_specs = [
        pl.BlockSpec((NC, blk, F), lambda i: (0, i, 0)),
        pl.BlockSpec((blk, F), lambda i: (i, 0)),
        pl.BlockSpec((blk, F), lambda i: (i, 0)),
        pl.BlockSpec((1, F), lambda i: (0, 0)),
        pl.BlockSpec((F, F), lambda i: (0, 0)),
    ]
  else:
    def body(accp_ref, g_ref, dinv_ref, b_ref, out_ref):
      z = (accp_ref[0] + accp_ref[1] + g_ref[...]) * dinv_ref[...]
      out_ref[...] = z + b_ref[...]
    in_specs = [
        pl.BlockSpec((NC, blk, F), lambda i: (0, i, 0)),
        pl.BlockSpec((blk, F), lambda i: (i, 0)),
        pl.BlockSpec((blk, F), lambda i: (i, 0)),
        pl.BlockSpec((1, F), lambda i: (0, 0)),
    ]

  return pl.pallas_call(
      body,
      grid=grid,
      in_specs=in_specs,
      out_specs=pl.BlockSpec((blk, F), lambda i: (i, 0)),
      out_shape=jax.ShapeDtypeStruct((n_nodes, F), jnp.float32),
  )


def _make_mlp_tc(n_rows, half_blocks, blk):
  grid = (n_rows // blk,)

  def body(zs_ref, zd_ref, l1_ref, b1_ref, l2_ref, b2_ref, l3_ref, b3_ref,
           out_ref):
    p = zs_ref[...] * zd_ref[...]
    a = jnp.maximum(_dot(p, l1_ref[...]) + b1_ref[...], 0.0)
    a = jnp.maximum(_dot(a, l2_ref[...]) + b2_ref[...], 0.0)
    out_ref[...] = _dot(a, l3_ref[...]) + b3_ref[...]

  return pl.pallas_call(
      body,
      grid=grid,
      in_specs=[
          pl.BlockSpec((blk, F), lambda i: (i, 0)),
          pl.BlockSpec((blk, F), lambda i: (i + half_blocks, 0)),
          pl.BlockSpec((F, F), lambda i: (0, 0)),
          pl.BlockSpec((1, F), lambda i: (0, 0)),
          pl.BlockSpec((F, F), lambda i: (0, 0)),
          pl.BlockSpec((1, F), lambda i: (0, 0)),
          pl.BlockSpec((F, 2), lambda i: (0, 0)),
          pl.BlockSpec((1, 2), lambda i: (0, 0)),
      ],
      out_specs=pl.BlockSpec((blk, 2), lambda i: (i, 0)),
      out_shape=jax.ShapeDtypeStruct((n_rows, 2), jnp.float32),
  )


def _pad_to(idx, total, fill):
  idx = idx.astype(jnp.int32)
  pad = total - idx.shape[0]
  if pad:
    idx = jnp.concatenate([idx, jnp.full((pad,), fill, jnp.int32)])
  return idx


def _pad_split_idx(idx, n_chunks, fill):
  total = NC * NS * n_chunks * ECHUNK
  return _pad_to(idx, total, fill).reshape(NC, NS, n_chunks, ECHUNK)


def _ceil_div(a, b):
  return -(-a // b)


def kernel(x, edge_index, edge_label_index,
           W1, b1, W2, b2, W3, b3, L1, bl1, L2, bl2, L3, bl3):
  n_nodes = x.shape[0]
  n_edges = edge_index.shape[1]
  n_label = edge_label_index.shape[1]

  n_pad = _ceil_div(n_nodes + 1, NS * 8) * NS * 8

  kround = NBUF * NPHASE
  ke = _ceil_div(_ceil_div(n_edges, NC * NS * ECHUNK), kround) * kround
  kl_half = _ceil_div(n_label, NC * NS * ECHUNK)
  half = NC * NS * kl_half * ECHUNK
  kl = 2 * kl_half
  n_gath = 2 * half

  src = _pad_split_idx(edge_index[0], ke, 0)
  dst = _pad_split_idx(edge_index[1], ke, n_pad - 1)
  lidx = jnp.concatenate([
      _pad_to(edge_label_index[0], half, 0),
      _pad_to(edge_label_index[1], half, 0),
  ]).reshape(NC, NS, kl, ECHUNK)

  zeros = jnp.zeros((n_pad, F), jnp.float32)
  ones = jnp.ones((ECHUNK, F), jnp.float32)
  b1r = b1.reshape(1, F)
  b2r = b2.reshape(1, F)
  b3r = b3.reshape(1, F)
  bl1r = bl1.reshape(1, F)
  bl2r = bl2.reshape(1, F)
  bl3r = bl3.reshape(1, 2)

  deg_k = _make_deg_kernel(n_pad, ke)
  scat_k = _make_edge_scatter_kernel(n_pad, ke)
  gath_k = _make_gather_kernel(n_gath, kl)

  blk = 1000
  layer1 = _make_layer1_tc(n_nodes, n_pad, blk)
  cmb_mm = _make_combine_tc(n_nodes, blk, True)
  cmb_fin = _make_combine_tc(n_nodes, blk, False)

  mlp_blk = 1024
  half_blocks = (n_gath // 2) // mlp_blk
  mlp = _make_mlp_tc(n_gath // 2, half_blocks, mlp_blk)

  degp = deg_k(dst, zeros, ones)
  dinv, g1 = layer1(degp, x, W1)

  acc1 = scat_k(g1, src, dst, zeros)
  g2 = cmb_mm(acc1, g1, dinv, b1r, W2)

  acc2 = scat_k(g2, src, dst, zeros)
  g3 = cmb_mm(acc2, g2, dinv, b2r, W3)

  acc3 = scat_k(g3, src, dst, zeros)
  z = cmb_fin(acc3, g3, dinv, b3r)

  zg = gath_k(z, lidx)
  logits = mlp(zg, zg, L1, bl1r, L2, bl2r, L3, bl3r)
  return logits[:n_label]


# speedup vs baseline: 1.2613x; 1.2613x over previous
import functools

import jax
import jax.numpy as jnp
from jax import lax
from jax.experimental import pallas as pl
from jax.experimental.pallas import tpu as pltpu
from jax.experimental.pallas import tpu_sc as plsc

NC = 2
NS = 16
ECHUNK = 128
F = 128


def _sc_mesh():
  return plsc.VectorSubcoreMesh(
      core_axis_name="c", subcore_axis_name="s",
      num_cores=NC, num_subcores=NS)


def _make_deg_kernel(n_pad, n_chunks):
  rows_per_sub = n_pad // NS

  @functools.partial(
      pl.kernel, mesh=_sc_mesh(),
      out_type=jax.ShapeDtypeStruct((NC, n_pad, F), jnp.float32),
      scratch_types=[
          pltpu.VMEM((n_chunks, ECHUNK), jnp.int32),
          pltpu.VMEM((ECHUNK, F), jnp.float32),
          pltpu.VMEM_SHARED((n_pad, F), jnp.float32),
      ],
  )
  def k(dsts_hbm, zeros_hbm, ones_hbm, out_hbm, dst_v, ones_v, acc_s):
    c = lax.axis_index("c")
    s = lax.axis_index("s")
    pltpu.sync_copy(dsts_hbm.at[c, s], dst_v)
    pltpu.sync_copy(ones_hbm, ones_v)
    sl = pl.ds(s * rows_per_sub, rows_per_sub)
    pltpu.sync_copy(zeros_hbm.at[sl], acc_s.at[sl])
    plsc.subcore_barrier()

    @pl.loop(0, n_chunks)
    def _(j):
      pltpu.sync_copy(ones_v, acc_s.at[dst_v.at[j]], add=True)

    plsc.subcore_barrier()
    pltpu.sync_copy(acc_s.at[sl], out_hbm.at[c, sl])

  return k


def _make_edge_scatter_kernel(n_pad, n_chunks):
  rows_per_sub = n_pad // NS

  @functools.partial(
      pl.kernel, mesh=_sc_mesh(),
      out_type=jax.ShapeDtypeStruct((NC, n_pad, F), jnp.float32),
      scratch_types=[
          pltpu.VMEM((n_chunks, ECHUNK), jnp.int32),
          pltpu.VMEM((n_chunks, ECHUNK), jnp.int32),
          pltpu.VMEM((ECHUNK, F), jnp.float32),
          pltpu.VMEM_SHARED((n_pad, F), jnp.float32),
      ],
  )
  def k(g_hbm, srcs_hbm, dsts_hbm, zeros_hbm, out_hbm,
        src_v, dst_v, buf_v, acc_s):
    c = lax.axis_index("c")
    s = lax.axis_index("s")
    pltpu.sync_copy(srcs_hbm.at[c, s], src_v)
    pltpu.sync_copy(dsts_hbm.at[c, s], dst_v)
    sl = pl.ds(s * rows_per_sub, rows_per_sub)
    pltpu.sync_copy(zeros_hbm.at[sl], acc_s.at[sl])
    plsc.subcore_barrier()

    @pl.loop(0, n_chunks)
    def _(j):
      pltpu.sync_copy(g_hbm.at[src_v.at[j]], buf_v)
      pltpu.sync_copy(buf_v, acc_s.at[dst_v.at[j]], add=True)

    plsc.subcore_barrier()
    pltpu.sync_copy(acc_s.at[sl], out_hbm.at[c, sl])

  return k


def _make_gather_kernel(n_rows_out, n_chunks):
  rows_per_tile = n_chunks * ECHUNK

  @functools.partial(
      pl.kernel, mesh=_sc_mesh(),
      out_type=jax.ShapeDtypeStruct((n_rows_out, F), jnp.float32),
      scratch_types=[
          pltpu.VMEM((n_chunks, ECHUNK), jnp.int32),
          pltpu.VMEM((ECHUNK, F), jnp.float32),
      ],
  )
  def k(z_hbm, idx_hbm, out_hbm, idx_v, buf_v):
    c = lax.axis_index("c")
    s = lax.axis_index("s")
    base = (c * NS + s) * rows_per_tile
    pltpu.sync_copy(idx_hbm.at[c, s], idx_v)

    @pl.loop(0, n_chunks)
    def _(j):
      pltpu.sync_copy(z_hbm.at[idx_v.at[j]], buf_v)
      pltpu.sync_copy(buf_v, out_hbm.at[pl.ds(base + j * ECHUNK, ECHUNK)])

  return k


def _dot(a, b):
  return jnp.dot(a, b, preferred_element_type=jnp.float32,
                 precision=lax.Precision.HIGHEST)


def _make_layer1_tc(n_nodes, n_pad, blk):
  grid = (n_nodes // blk,)

  def body(degp_ref, x_ref, w_ref, dinv_ref, g_ref):
    deg = degp_ref[0] + degp_ref[1] + 1.0
    dinv = lax.rsqrt(deg)
    h = _dot(x_ref[...], w_ref[...])
    dinv_ref[...] = dinv
    g_ref[...] = h * dinv

  return pl.pallas_call(
      body,
      grid=grid,
      in_specs=[
          pl.BlockSpec((NC, blk, F), lambda i: (0, i, 0)),
          pl.BlockSpec((blk, F), lambda i: (i, 0)),
          pl.BlockSpec((F, F), lambda i: (0, 0)),
      ],
      out_specs=[
          pl.BlockSpec((blk, F), lambda i: (i, 0)),
          pl.BlockSpec((blk, F), lambda i: (i, 0)),
      ],
      out_shape=[
          jax.ShapeDtypeStruct((n_nodes, F), jnp.float32),
          jax.ShapeDtypeStruct((n_nodes, F), jnp.float32),
      ],
  )


def _make_combine_tc(n_nodes, blk, with_matmul):
  grid = (n_nodes // blk,)

  if with_matmul:
    def body(accp_ref, g_ref, dinv_ref, b_ref, w_ref, out_ref):
      dinv = dinv_ref[...]
      z = (accp_ref[0] + accp_ref[1] + g_ref[...]) * dinv + b_ref[...]
      z = jnp.maximum(z, 0.0)
      out_ref[...] = _dot(z, w_ref[...]) * dinv
    in_specs = [
        pl.BlockSpec((NC, blk, F), lambda i: (0, i, 0)),
        pl.BlockSpec((blk, F), lambda i: (i, 0)),
        pl.BlockSpec((blk, F), lambda i: (i, 0)),
        pl.BlockSpec((1, F), lambda i: (0, 0)),
        pl.BlockSpec((F, F), lambda i: (0, 0)),
    ]
  else:
    def body(accp_ref, g_ref, dinv_ref, b_ref, out_ref):
      z = (accp_ref[0] + accp_ref[1] + g_ref[...]) * dinv_ref[...]
      out_ref[...] = z + b_ref[...]
    in_specs = [
        pl.BlockSpec((NC, blk, F), lambda i: (0, i, 0)),
        pl.BlockSpec((blk, F), lambda i: (i, 0)),
        pl.BlockSpec((blk, F), lambda i: (i, 0)),
        pl.BlockSpec((1, F), lambda i: (0, 0)),
    ]

  return pl.pallas_call(
      body,
      grid=grid,
      in_specs=in_specs,
      out_specs=pl.BlockSpec((blk, F), lambda i: (i, 0)),
      out_shape=jax.ShapeDtypeStruct((n_nodes, F), jnp.float32),
  )


def _make_mlp_tc(n_rows, half_blocks, blk):
  grid = (n_rows // blk,)

  def body(zs_ref, zd_ref, l1_ref, b1_ref, l2_ref, b2_ref, l3_ref, b3_ref,
           out_ref):
    p = zs_ref[...] * zd_ref[...]
    a = jnp.maximum(_dot(p, l1_ref[...]) + b1_ref[...], 0.0)
    a = jnp.maximum(_dot(a, l2_ref[...]) + b2_ref[...], 0.0)
    out_ref[...] = _dot(a, l3_ref[...]) + b3_ref[...]

  return pl.pallas_call(
      body,
      grid=grid,
      in_specs=[
          pl.BlockSpec((blk, F), lambda i: (i, 0)),
          pl.BlockSpec((blk, F), lambda i: (i + half_blocks, 0)),
          pl.BlockSpec((F, F), lambda i: (0, 0)),
          pl.BlockSpec((1, F), lambda i: (0, 0)),
          pl.BlockSpec((F, F), lambda i: (0, 0)),
          pl.BlockSpec((1, F), lambda i: (0, 0)),
          pl.BlockSpec((F, 2), lambda i: (0, 0)),
          pl.BlockSpec((1, 2), lambda i: (0, 0)),
      ],
      out_specs=pl.BlockSpec((blk, 2), lambda i: (i, 0)),
      out_shape=jax.ShapeDtypeStruct((n_rows, 2), jnp.float32),
  )


def _pad_to(idx, total, fill):
  idx = idx.astype(jnp.int32)
  pad = total - idx.shape[0]
  if pad:
    idx = jnp.concatenate([idx, jnp.full((pad,), fill, jnp.int32)])
  return idx


def _pad_split_idx(idx, n_chunks, fill):
  total = NC * NS * n_chunks * ECHUNK
  return _pad_to(idx, total, fill).reshape(NC, NS, n_chunks, ECHUNK)


def _ceil_div(a, b):
  return -(-a // b)


def kernel(x, edge_index, edge_label_index,
           W1, b1, W2, b2, W3, b3, L1, bl1, L2, bl2, L3, bl3):
  n_nodes = x.shape[0]
  n_edges = edge_index.shape[1]
  n_label = edge_label_index.shape[1]

  n_pad = _ceil_div(n_nodes + 1, NS * 8) * NS * 8

  ke = _ceil_div(n_edges, NC * NS * ECHUNK)
  kl_half = _ceil_div(n_label, NC * NS * ECHUNK)
  half = NC * NS * kl_half * ECHUNK
  kl = 2 * kl_half
  n_gath = 2 * half

  src = _pad_split_idx(edge_index[0], ke, 0)
  dst = _pad_split_idx(edge_index[1], ke, n_pad - 1)
  lidx = jnp.concatenate([
      _pad_to(edge_label_index[0], half, 0),
      _pad_to(edge_label_index[1], half, 0),
  ]).reshape(NC, NS, kl, ECHUNK)

  zeros = jnp.zeros((n_pad, F), jnp.float32)
  ones = jnp.ones((ECHUNK, F), jnp.float32)
  b1r = b1.reshape(1, F)
  b2r = b2.reshape(1, F)
  b3r = b3.reshape(1, F)
  bl1r = bl1.reshape(1, F)
  bl2r = bl2.reshape(1, F)
  bl3r = bl3.reshape(1, 2)

  deg_k = _make_deg_kernel(n_pad, ke)
  scat_k = _make_edge_scatter_kernel(n_pad, ke)
  gath_k = _make_gather_kernel(n_gath, kl)

  blk = 1000
  layer1 = _make_layer1_tc(n_nodes, n_pad, blk)
  cmb_mm = _make_combine_tc(n_nodes, blk, True)
  cmb_fin = _make_combine_tc(n_nodes, blk, False)

  mlp_blk = 1024
  half_blocks = (n_gath // 2) // mlp_blk
  mlp = _make_mlp_tc(n_gath // 2, half_blocks, mlp_blk)

  degp = deg_k(dst, zeros, ones)
  dinv, g1 = layer1(degp, x, W1)

  acc1 = scat_k(g1, src, dst, zeros)
  g2 = cmb_mm(acc1, g1, dinv, b1r, W2)

  acc2 = scat_k(g2, src, dst, zeros)
  g3 = cmb_mm(acc2, g2, dinv, b2r, W3)

  acc3 = scat_k(g3, src, dst, zeros)
  z = cmb_fin(acc3, g3, dinv, b3r)

  zg = gath_k(z, lidx)
  logits = mlp(zg, zg, L1, bl1r, L2, bl2r, L3, bl3r)
  return logits[:n_label]
